# baseline (speedup 1.0000x reference)
import functools

import jax
import jax.numpy as jnp
from jax import lax
from jax.experimental import pallas as pl
from jax.experimental.pallas import tpu as pltpu
from jax.experimental.pallas import tpu_sc as plsc

NB, NT = 16384, 50
D = 64
TPAD = 56
IPAD = 64
NC, NS = 2, 16
NW = NC * NS
B_PER_W = NB // NW
G = 4
N_SLOTS = B_PER_W // G
NBUF = 3
LAG = 1

_mesh = plsc.VectorSubcoreMesh(core_axis_name="c", subcore_axis_name="s")


@functools.partial(
    pl.kernel,
    mesh=_mesh,
    compiler_params=pltpu.CompilerParams(use_tc_tiling_on_sc=False),
    out_type=jax.ShapeDtypeStruct((NB, TPAD, 2 * D), jnp.float32),
    scratch_types=(
        [pltpu.VMEM((B_PER_W, IPAD), jnp.int32)]
        + [pltpu.VMEM((G, TPAD, 2 * D), jnp.float32) for _ in range(NBUF)]
        + [pltpu.SemaphoreType.DMA for _ in range(2 * NBUF)]
    ),
)
def _emb_lookup(idx_hbm, tab_hbm, out_hbm, idx_v, *scratch):
    rows = scratch[:NBUF]
    gsem = scratch[NBUF:2 * NBUF]
    wsem = scratch[2 * NBUF:]
    wid = lax.axis_index("s") * NC + lax.axis_index("c")
    base = wid * B_PER_W
    pltpu.sync_copy(idx_hbm.at[wid], idx_v)

    def g_issue(j, b):
        for i in range(G):
            pltpu.async_copy(
                tab_hbm.at[idx_v.at[j * G + i, pl.ds(0, TPAD)]],
                rows[b].at[i],
                gsem[b],
            )

    def g_wait(b):
        pltpu.make_async_copy(
            out_hbm.at[pl.ds(0, G)], rows[b], gsem[b]
        ).wait()

    def w_issue(j, b):
        pltpu.async_copy(
            rows[b],
            out_hbm.at[pl.ds(base + j * G, G)],
            wsem[b],
        )

    def w_wait(b):
        pltpu.make_async_copy(
            rows[b], out_hbm.at[pl.ds(0, G)], wsem[b]
        ).wait()

    def group(gi, carry):
        for b in range(NBUF):
            j = gi * NBUF + b

            @pl.when(j < N_SLOTS)
            def _():
                @pl.when(j >= NBUF)
                def _():
                    w_wait(b)
                g_issue(j, b)

            jw = j - LAG
            bw = (b - LAG) % NBUF

            @pl.when((jw >= 0) & (jw < N_SLOTS))
            def _():
                g_wait(bw)
                w_issue(jw, bw)

        return carry

    n_groups = (N_SLOTS + LAG + NBUF - 1) // NBUF
    lax.fori_loop(0, n_groups, group, 0)
    for b in range(NBUF):
        w_wait(b)


def kernel(token_ids, weight):
    idx = jnp.pad(token_ids.astype(jnp.int32), ((0, 0), (0, IPAD - NT)))
    idx = idx.reshape(NW, B_PER_W, IPAD)
    wpad = jnp.pad(weight, ((0, 0), (0, D)))
    out = _emb_lookup(idx, wpad)
    return out[:, :NT, :D]


# speedup vs baseline: 1.0005x; 1.0005x over previous
import functools

import jax
import jax.numpy as jnp
from jax import lax
from jax.experimental import pallas as pl
from jax.experimental.pallas import tpu as pltpu
from jax.experimental.pallas import tpu_sc as plsc

NB, NT = 16384, 50
D = 64
TPAD = 56
IPAD = 64
NC, NS = 2, 16
NW = NC * NS
B_PER_W = NB // NW
G = 4
N_SLOTS = B_PER_W // G
NBUF = 3
LAG = 1

_mesh = plsc.VectorSubcoreMesh(core_axis_name="c", subcore_axis_name="s")


@functools.partial(
    pl.kernel,
    mesh=_mesh,
    compiler_params=pltpu.CompilerParams(use_tc_tiling_on_sc=False),
    out_type=jax.ShapeDtypeStruct((NB, TPAD, 2 * D), jnp.float32),
    scratch_types=(
        [pltpu.VMEM((B_PER_W, TPAD), jnp.int32)]
        + [pltpu.VMEM((G, TPAD, 2 * D), jnp.float32) for _ in range(NBUF)]
        + [pltpu.SemaphoreType.DMA for _ in range(2 * NBUF)]
    ),
)
def _emb_lookup(idx_hbm, tab_hbm, out_hbm, idx_v, *scratch):
    rows = scratch[:NBUF]
    gsem = scratch[NBUF:2 * NBUF]
    wsem = scratch[2 * NBUF:]
    wid = lax.axis_index("s") * NC + lax.axis_index("c")
    base = wid * B_PER_W
    pltpu.sync_copy(idx_hbm.at[wid, pl.ds(0, B_PER_W), pl.ds(0, TPAD)], idx_v)

    def g_issue(j, b):
        for i in range(G):
            pltpu.async_copy(
                tab_hbm.at[idx_v.at[j * G + i]],
                rows[b].at[i],
                gsem[b],
            )

    def g_wait(b):
        pltpu.make_async_copy(
            out_hbm.at[pl.ds(0, G)], rows[b], gsem[b]
        ).wait()

    def w_issue(j, b):
        pltpu.async_copy(
            rows[b],
            out_hbm.at[pl.ds(base + j * G, G)],
            wsem[b],
        )

    def w_wait(b):
        pltpu.make_async_copy(
            rows[b], out_hbm.at[pl.ds(0, G)], wsem[b]
        ).wait()

    def group(gi, carry):
        for b in range(NBUF):
            j = gi * NBUF + b

            @pl.when(j < N_SLOTS)
            def _():
                @pl.when(j >= NBUF)
                def _():
                    w_wait(b)
                g_issue(j, b)

            jw = j - LAG
            bw = (b - LAG) % NBUF

            @pl.when((jw >= 0) & (jw < N_SLOTS))
            def _():
                g_wait(bw)
                w_issue(jw, bw)

        return carry

    n_groups = (N_SLOTS + LAG + NBUF - 1) // NBUF
    lax.fori_loop(0, n_groups, group, 0)
    for b in range(NBUF):
        w_wait(b)


def kernel(token_ids, weight):
    idx = jnp.pad(token_ids.astype(jnp.int32), ((0, 0), (0, IPAD - NT)))
    idx = idx.reshape(NW, B_PER_W, IPAD)
    wpad = jnp.pad(weight, ((0, 0), (0, D)))
    out = _emb_lookup(idx, wpad)
    return out[:, :NT, :D]


# speedup vs baseline: 1.7262x; 1.7253x over previous
import functools

import jax
import jax.numpy as jnp
from jax import lax
from jax.experimental import pallas as pl
from jax.experimental.pallas import tpu as pltpu
from jax.experimental.pallas import tpu_sc as plsc

NB, NT = 16384, 50
D = 64
TPAD = 56
IPAD = 64
NC, NS = 2, 16
NW = NC * NS
B_PER_W = NB // NW
G = 4
N_SLOTS = B_PER_W // G
NBUF = 3
LAG = 1

_mesh = plsc.VectorSubcoreMesh(core_axis_name="c", subcore_axis_name="s")


@functools.partial(
    pl.kernel,
    mesh=_mesh,
    compiler_params=pltpu.CompilerParams(use_tc_tiling_on_sc=False),
    out_type=jax.ShapeDtypeStruct((NB, TPAD, 2 * D), jnp.float32),
    scratch_types=(
        [pltpu.VMEM((B_PER_W, TPAD), jnp.int32)]
        + [pltpu.VMEM((G, TPAD, D), jnp.float32) for _ in range(NBUF)]
        + [pltpu.SemaphoreType.DMA for _ in range(2 * NBUF)]
    ),
)
def _emb_lookup(idx_hbm, tab_hbm, out_hbm, idx_v, *scratch):
    rows = scratch[:NBUF]
    gsem = scratch[NBUF:2 * NBUF]
    wsem = scratch[2 * NBUF:]
    wid = lax.axis_index("s") * NC + lax.axis_index("c")
    base = wid * B_PER_W
    pltpu.sync_copy(idx_hbm.at[wid, pl.ds(0, B_PER_W), pl.ds(0, TPAD)], idx_v)

    def g_issue(j, b):
        for i in range(G):
            pltpu.async_copy(
                tab_hbm.at[idx_v.at[j * G + i]],
                rows[b].at[i],
                gsem[b],
            )

    def g_wait(b):
        pltpu.make_async_copy(
            out_hbm.at[pl.ds(0, G), pl.ds(0, TPAD), pl.ds(0, D)], rows[b], gsem[b]
        ).wait()

    def w_issue(j, b):
        pltpu.async_copy(
            rows[b],
            out_hbm.at[pl.ds(base + j * G, G), pl.ds(0, TPAD), pl.ds(0, D)],
            wsem[b],
        )

    def w_wait(b):
        pltpu.make_async_copy(
            rows[b],
            out_hbm.at[pl.ds(0, G), pl.ds(0, TPAD), pl.ds(0, D)],
            wsem[b],
        ).wait()

    def group(gi, carry):
        for b in range(NBUF):
            j = gi * NBUF + b

            @pl.when(j < N_SLOTS)
            def _():
                @pl.when(j >= NBUF)
                def _():
                    w_wait(b)
                g_issue(j, b)

            jw = j - LAG
            bw = (b - LAG) % NBUF

            @pl.when((jw >= 0) & (jw < N_SLOTS))
            def _():
                g_wait(bw)
                w_issue(jw, bw)

        return carry

    n_groups = (N_SLOTS + LAG + NBUF - 1) // NBUF
    lax.fori_loop(0, n_groups, group, 0)
    for b in range(NBUF):
        w_wait(b)


def kernel(token_ids, weight):
    idx = jnp.pad(token_ids.astype(jnp.int32), ((0, 0), (0, IPAD - NT)))
    idx = idx.reshape(NW, B_PER_W, IPAD)
    out = _emb_lookup(idx, weight)
    return out[:, :NT, :D]


# speedup vs baseline: 4.3045x; 2.4936x over previous
import functools

import jax
import jax.numpy as jnp
from jax import lax
from jax.experimental import pallas as pl
from jax.experimental.pallas import tpu as pltpu
from jax.experimental.pallas import tpu_sc as plsc

D = 64
B = 16384 * 50
NC, NS = 2, 16
NW = NC * NS
B_PER_W = B // NW
CHUNK = 128
WCHUNK = 256
SPB = WCHUNK // CHUNK
N_SLOTS = B_PER_W // WCHUNK
NBUF = 5
LAG = 2

_mesh = plsc.VectorSubcoreMesh(core_axis_name="c", subcore_axis_name="s")


@functools.partial(
    pl.kernel,
    mesh=_mesh,
    compiler_params=pltpu.CompilerParams(use_tc_tiling_on_sc=False),
    out_type=jax.ShapeDtypeStruct((B, D), jnp.float32),
    scratch_types=(
        [pltpu.VMEM((B_PER_W // CHUNK, CHUNK), jnp.int32)]
        + [pltpu.VMEM((WCHUNK, D), jnp.float32) for _ in range(NBUF)]
        + [pltpu.SemaphoreType.DMA for _ in range(2 * NBUF)]
    ),
)
def _emb_lookup(idx_hbm, tab_hbm, out_hbm, idx_v, *scratch):
    rows = scratch[:NBUF]
    gsem = scratch[NBUF:2 * NBUF]
    wsem = scratch[2 * NBUF:]
    wid = lax.axis_index("s") * NC + lax.axis_index("c")
    base = wid * B_PER_W
    pltpu.sync_copy(idx_hbm.at[wid], idx_v)

    def g_issue(j, b):
        for h in range(SPB):
            pltpu.async_copy(
                tab_hbm.at[idx_v.at[j * SPB + h]],
                rows[b].at[pl.ds(h * CHUNK, CHUNK)],
                gsem[b],
            )

    def g_wait(b):
        pltpu.make_async_copy(out_hbm.at[pl.ds(0, WCHUNK)], rows[b], gsem[b]).wait()

    def w_issue(j, b):
        pltpu.async_copy(rows[b], out_hbm.at[pl.ds(base + j * WCHUNK, WCHUNK)], wsem[b])

    def w_wait(b):
        pltpu.make_async_copy(rows[b], out_hbm.at[pl.ds(0, WCHUNK)], wsem[b]).wait()

    def group(gi, carry):
        for b in range(NBUF):
            j = gi * NBUF + b

            @pl.when(j < N_SLOTS)
            def _():
                @pl.when(j >= NBUF)
                def _():
                    w_wait(b)
                g_issue(j, b)

            jw = j - LAG
            bw = (b - LAG) % NBUF

            @pl.when((jw >= 0) & (jw < N_SLOTS))
            def _():
                g_wait(bw)
                w_issue(jw, bw)

        return carry

    n_groups = (N_SLOTS + LAG + NBUF - 1) // NBUF
    lax.fori_loop(0, n_groups, group, 0)
    for b in range(NBUF):
        w_wait(b)


def kernel(token_ids, weight):
    idx = token_ids.reshape(NW, B_PER_W // CHUNK, CHUNK).astype(jnp.int32)
    w_lin = jax.lax.optimization_barrier(weight.reshape(-1))
    out = _emb_lookup(idx, w_lin.reshape(weight.shape))
    out_lin = jax.lax.optimization_barrier(out.reshape(-1))
    return out_lin.reshape(token_ids.shape + (D,))
